# baseline (speedup 1.0000x reference)
import jax
import jax.numpy as jnp
from jax.experimental import pallas as pl
from jax.experimental.pallas import tpu as pltpu

LANE = 128
VMEM_LIMIT = 48 << 20
BF = jnp.bfloat16


def _round_up(a, m):
    return (a + m - 1) // m * m


def _pass1_kernel(g_ref, xb_ref, w1_ref, b1_ref, w2_ref, b2_ref,
                  x1_ref, h2_ref, gb_ref):
    g = g_ref[...]
    gb = g.astype(BF)
    gb_ref[...] = gb
    y = jnp.dot(gb, xb_ref[...], preferred_element_type=jnp.float32)
    s = jnp.sum(g, axis=1, keepdims=True)
    x1 = (jnp.dot(y, w1_ref[...], preferred_element_type=jnp.float32)
          + s * b1_ref[...])
    x1_ref[...] = x1
    h2_ref[...] = (jnp.dot(x1, w2_ref[...],
                           preferred_element_type=jnp.float32)
                   + b2_ref[...]).astype(BF)


def _pass2_kernel(gb_ref, h2_ref, x1_ref, out_ref):
    out_ref[...] = (jnp.dot(gb_ref[...], h2_ref[...],
                            preferred_element_type=jnp.float32)
                    + x1_ref[...]) * 0.5


def kernel(x, G, w1, b1, w2, b2):
    N, H = x.shape
    dt = x.dtype
    Hp = _round_up(max(H, LANE), LANE)
    TM = 512
    Np = _round_up(N, TM)

    if (Np, Hp) == (N, H):
        x_p, g_p, w1_p, w2_p = x, G, w1, w2
        b1_p = b1.reshape(1, H)
        b2_p = b2.reshape(1, H)
    else:
        x_p = jnp.zeros((Np, Hp), dt).at[:N, :H].set(x)
        g_p = jnp.zeros((Np, Np), dt).at[:N, :N].set(G)
        w1_p = jnp.zeros((Hp, Hp), dt).at[:H, :H].set(w1)
        w2_p = jnp.zeros((Hp, Hp), dt).at[:H, :H].set(w2)
        b1_p = jnp.zeros((1, Hp), dt).at[0, :H].set(b1)
        b2_p = jnp.zeros((1, Hp), dt).at[0, :H].set(b2)

    xb = x_p.astype(BF)

    grid = (Np // TM,)
    row_spec = pl.BlockSpec((TM, Hp), lambda i: (i, 0))
    g_spec = pl.BlockSpec((TM, Np), lambda i: (i, 0))
    mat_spec = pl.BlockSpec((Hp, Hp), lambda i: (0, 0))
    vec_spec = pl.BlockSpec((1, Hp), lambda i: (0, 0))
    full_spec = pl.BlockSpec((Np, Hp), lambda i: (0, 0))
    params = pltpu.CompilerParams(
        dimension_semantics=("parallel",),
        vmem_limit_bytes=VMEM_LIMIT,
    )

    x1, h2, gb = pl.pallas_call(
        _pass1_kernel,
        out_shape=(jax.ShapeDtypeStruct((Np, Hp), jnp.float32),
                   jax.ShapeDtypeStruct((Np, Hp), BF),
                   jax.ShapeDtypeStruct((Np, Np), BF)),
        grid=grid,
        in_specs=[g_spec, full_spec, mat_spec, vec_spec, mat_spec, vec_spec],
        out_specs=(row_spec, row_spec, g_spec),
        compiler_params=params,
        cost_estimate=pl.CostEstimate(
            flops=2 * Np * Np * Hp + 4 * Np * Hp * Hp,
            transcendentals=0,
            bytes_accessed=(Np * Np + 3 * Np * Hp) * 4,
        ),
    )(g_p, xb, w1_p, b1_p, w2_p, b2_p)

    out = pl.pallas_call(
        _pass2_kernel,
        out_shape=jax.ShapeDtypeStruct((Np, Hp), jnp.float32),
        grid=grid,
        in_specs=[g_spec, full_spec, row_spec],
        out_specs=row_spec,
        compiler_params=params,
        cost_estimate=pl.CostEstimate(
            flops=2 * Np * Np * Hp,
            transcendentals=0,
            bytes_accessed=(Np * Np * 2 + 3 * Np * Hp * 4),
        ),
    )(gb, h2, x1)

    if (Np, Hp) == (N, H):
        return out
    return out[:N, :H]


# speedup vs baseline: 1.0725x; 1.0725x over previous
import jax
import jax.numpy as jnp
from jax.experimental import pallas as pl
from jax.experimental.pallas import tpu as pltpu

LANE = 128
VMEM_LIMIT = 48 << 20
BF = jnp.bfloat16


def _round_up(a, m):
    return (a + m - 1) // m * m


def _pass1_kernel(g_ref, xb_ref, w1_ref, b1_ref, w2_ref, b2_ref,
                  x1_ref, h2_ref):
    g = g_ref[...]
    y = jnp.dot(g.astype(BF), xb_ref[...],
                preferred_element_type=jnp.float32)
    s = jnp.sum(g, axis=1, keepdims=True)
    x1 = (jnp.dot(y, w1_ref[...], preferred_element_type=jnp.float32)
          + s * b1_ref[...])
    x1_ref[...] = x1.astype(BF)
    h2_ref[...] = (jnp.dot(x1, w2_ref[...],
                           preferred_element_type=jnp.float32)
                   + b2_ref[...]).astype(BF)


def _pass2_kernel(g_ref, h2_ref, x1_ref, out_ref):
    out_ref[...] = (jnp.dot(g_ref[...].astype(BF), h2_ref[...],
                            preferred_element_type=jnp.float32)
                    + x1_ref[...].astype(jnp.float32)) * 0.5


def kernel(x, G, w1, b1, w2, b2):
    N, H = x.shape
    dt = x.dtype
    Hp = _round_up(max(H, LANE), LANE)
    TM = 512
    Np = _round_up(N, TM)

    if (Np, Hp) == (N, H):
        x_p, g_p, w1_p, w2_p = x, G, w1, w2
        b1_p = b1.reshape(1, H)
        b2_p = b2.reshape(1, H)
    else:
        x_p = jnp.zeros((Np, Hp), dt).at[:N, :H].set(x)
        g_p = jnp.zeros((Np, Np), dt).at[:N, :N].set(G)
        w1_p = jnp.zeros((Hp, Hp), dt).at[:H, :H].set(w1)
        w2_p = jnp.zeros((Hp, Hp), dt).at[:H, :H].set(w2)
        b1_p = jnp.zeros((1, Hp), dt).at[0, :H].set(b1)
        b2_p = jnp.zeros((1, Hp), dt).at[0, :H].set(b2)

    xb = x_p.astype(BF)

    grid = (Np // TM,)
    row_spec = pl.BlockSpec((TM, Hp), lambda i: (i, 0))
    g_spec = pl.BlockSpec((TM, Np), lambda i: (i, 0))
    mat_spec = pl.BlockSpec((Hp, Hp), lambda i: (0, 0))
    vec_spec = pl.BlockSpec((1, Hp), lambda i: (0, 0))
    full_spec = pl.BlockSpec((Np, Hp), lambda i: (0, 0))
    params = pltpu.CompilerParams(
        dimension_semantics=("parallel",),
        vmem_limit_bytes=VMEM_LIMIT,
    )

    x1, h2 = pl.pallas_call(
        _pass1_kernel,
        out_shape=(jax.ShapeDtypeStruct((Np, Hp), BF),
                   jax.ShapeDtypeStruct((Np, Hp), BF)),
        grid=grid,
        in_specs=[g_spec, full_spec, mat_spec, vec_spec, mat_spec, vec_spec],
        out_specs=(row_spec, row_spec),
        compiler_params=params,
        cost_estimate=pl.CostEstimate(
            flops=2 * Np * Np * Hp + 4 * Np * Hp * Hp,
            transcendentals=0,
            bytes_accessed=(Np * Np + 3 * Np * Hp) * 4,
        ),
    )(g_p, xb, w1_p, b1_p, w2_p, b2_p)

    out = pl.pallas_call(
        _pass2_kernel,
        out_shape=jax.ShapeDtypeStruct((Np, Hp), jnp.float32),
        grid=grid,
        in_specs=[g_spec, full_spec, row_spec],
        out_specs=row_spec,
        compiler_params=params,
        cost_estimate=pl.CostEstimate(
            flops=2 * Np * Np * Hp,
            transcendentals=0,
            bytes_accessed=(Np * Np + 3 * Np * Hp) * 4,
        ),
    )(g_p, h2, x1)

    if (Np, Hp) == (N, H):
        return out
    return out[:N, :H]


# speedup vs baseline: 1.0735x; 1.0009x over previous
import jax
import jax.numpy as jnp
from jax.experimental import pallas as pl
from jax.experimental.pallas import tpu as pltpu

LANE = 128
VMEM_LIMIT = 48 << 20
BF = jnp.bfloat16


def _round_up(a, m):
    return (a + m - 1) // m * m


def _pass1_kernel(g_ref, xb_ref, w1_ref, b1_ref, w2_ref, b2_ref,
                  x1_ref, h2_ref):
    g = g_ref[...]
    y = jnp.dot(g.astype(BF), xb_ref[...],
                preferred_element_type=jnp.float32)
    s = jnp.sum(g, axis=1, keepdims=True)
    x1 = (jnp.dot(y, w1_ref[...], preferred_element_type=jnp.float32)
          + s * b1_ref[...])
    x1_ref[...] = x1
    h2_ref[...] = (jnp.dot(x1, w2_ref[...],
                           preferred_element_type=jnp.float32)
                   + b2_ref[...]).astype(BF)


def _pass2_kernel(g_ref, h2_ref, x1_ref, out_ref):
    out_ref[...] = (jnp.dot(g_ref[...].astype(BF), h2_ref[...],
                            preferred_element_type=jnp.float32)
                    + x1_ref[...]) * 0.5


def kernel(x, G, w1, b1, w2, b2):
    N, H = x.shape
    dt = x.dtype
    Hp = _round_up(max(H, LANE), LANE)
    TM = 512
    Np = _round_up(N, TM)

    if (Np, Hp) == (N, H):
        x_p, g_p, w1_p, w2_p = x, G, w1, w2
        b1_p = b1.reshape(1, H)
        b2_p = b2.reshape(1, H)
    else:
        x_p = jnp.zeros((Np, Hp), dt).at[:N, :H].set(x)
        g_p = jnp.zeros((Np, Np), dt).at[:N, :N].set(G)
        w1_p = jnp.zeros((Hp, Hp), dt).at[:H, :H].set(w1)
        w2_p = jnp.zeros((Hp, Hp), dt).at[:H, :H].set(w2)
        b1_p = jnp.zeros((1, Hp), dt).at[0, :H].set(b1)
        b2_p = jnp.zeros((1, Hp), dt).at[0, :H].set(b2)

    xb = x_p.astype(BF)

    grid = (Np // TM,)
    row_spec = pl.BlockSpec((TM, Hp), lambda i: (i, 0))
    g_spec = pl.BlockSpec((TM, Np), lambda i: (i, 0))
    mat_spec = pl.BlockSpec((Hp, Hp), lambda i: (0, 0))
    vec_spec = pl.BlockSpec((1, Hp), lambda i: (0, 0))
    full_spec = pl.BlockSpec((Np, Hp), lambda i: (0, 0))
    params = pltpu.CompilerParams(
        dimension_semantics=("parallel",),
        vmem_limit_bytes=VMEM_LIMIT,
    )

    x1, h2 = pl.pallas_call(
        _pass1_kernel,
        out_shape=(jax.ShapeDtypeStruct((Np, Hp), jnp.float32),
                   jax.ShapeDtypeStruct((Np, Hp), BF)),
        grid=grid,
        in_specs=[g_spec, full_spec, mat_spec, vec_spec, mat_spec, vec_spec],
        out_specs=(row_spec, row_spec),
        compiler_params=params,
        cost_estimate=pl.CostEstimate(
            flops=2 * Np * Np * Hp + 4 * Np * Hp * Hp,
            transcendentals=0,
            bytes_accessed=(Np * Np + 3 * Np * Hp) * 4,
        ),
    )(g_p, xb, w1_p, b1_p, w2_p, b2_p)

    out = pl.pallas_call(
        _pass2_kernel,
        out_shape=jax.ShapeDtypeStruct((Np, Hp), jnp.float32),
        grid=grid,
        in_specs=[g_spec, full_spec, row_spec],
        out_specs=row_spec,
        compiler_params=params,
        cost_estimate=pl.CostEstimate(
            flops=2 * Np * Np * Hp,
            transcendentals=0,
            bytes_accessed=(Np * Np + 3 * Np * Hp) * 4,
        ),
    )(g_p, h2, x1)

    if (Np, Hp) == (N, H):
        return out
    return out[:N, :H]
